# baseline (speedup 1.0000x reference)
import functools

import jax
import jax.numpy as jnp
from jax import lax
from jax.experimental import pallas as pl
from jax.experimental.pallas import tpu as pltpu
from jax.experimental.pallas import tpu_sc as plsc

EPS = 1e-12

_NC = 2
_NS = 16
_NW = _NC * _NS

_CHUNK = 64

_TB = 256


def _sc_gather(table, ids):
    n_tok = ids.shape[0]
    hidden = table.shape[1]
    per_w = n_tok // _NW
    n_chunks = per_w // _CHUNK

    mesh = plsc.VectorSubcoreMesh(core_axis_name="c", subcore_axis_name="s")

    @functools.partial(
        pl.kernel,
        mesh=mesh,
        out_type=jax.ShapeDtypeStruct((n_tok, hidden), jnp.float32),
        scratch_types=[
            pltpu.VMEM((per_w,), jnp.int32),
            pltpu.VMEM((_CHUNK, hidden), jnp.float32),
            pltpu.VMEM((_CHUNK, hidden), jnp.float32),
            pltpu.SemaphoreType.DMA,
            pltpu.SemaphoreType.DMA,
        ],
    )
    def gather_kernel(table_hbm, ids_hbm, out_hbm, idx_v, buf0, buf1, sem0, sem1):
        wid = lax.axis_index("s") * _NC + lax.axis_index("c")
        base = wid * per_w
        pltpu.sync_copy(ids_hbm.at[pl.ds(base, per_w)], idx_v)
        bufs = (buf0, buf1)
        sems = (sem0, sem1)
        copies = [None] * n_chunks
        copies[0] = pltpu.async_copy(
            table_hbm.at[idx_v.at[pl.ds(0, _CHUNK)]], buf0, sem0
        )
        for k in range(n_chunks):
            nxt = k + 1
            if nxt < n_chunks:
                copies[nxt] = pltpu.async_copy(
                    table_hbm.at[idx_v.at[pl.ds(nxt * _CHUNK, _CHUNK)]],
                    bufs[nxt % 2],
                    sems[nxt % 2],
                )
            copies[k].wait()
            pltpu.sync_copy(bufs[k % 2], out_hbm.at[pl.ds(base + k * _CHUNK, _CHUNK)])

    return gather_kernel(table, ids)


def _ln_body(buf_ref, g_ref, tt_ref, pos_ref, tte_ref, w_ref, b_ref, o_ref):
    del buf_ref
    x = g_ref[...] + pos_ref[...]
    ttf = tt_ref[0, 0, :].astype(jnp.float32)
    t0 = tte_ref[0, :]
    t1 = tte_ref[1, :]
    x = x + t0[None, :] + ttf[:, None] * (t1 - t0)[None, :]
    u = jnp.mean(x, axis=-1, keepdims=True)
    s = jnp.mean((x - u) ** 2, axis=-1, keepdims=True)
    y = (x - u) * lax.rsqrt(s + EPS)
    o_ref[...] = y * w_ref[0, :][None, :] + b_ref[0, :][None, :]


def _tc_add_ln_slice(buf, gathered, tt_ids, pos_emb, tt_emb, ln_w, ln_b, n_tok, slice_idx):
    seq, hidden = gathered.shape
    nb = seq // _TB
    blk0 = slice_idx * nb

    tt3 = tt_ids.reshape(nb, 1, _TB)

    return pl.pallas_call(
        _ln_body,
        grid=(nb,),
        in_specs=[
            pl.BlockSpec(memory_space=pl.ANY),
            pl.BlockSpec((_TB, hidden), lambda i: (i, 0)),
            pl.BlockSpec((1, 1, _TB), lambda i: (i, 0, 0)),
            pl.BlockSpec((_TB, hidden), lambda i: (i, 0)),
            pl.BlockSpec((2, hidden), lambda i: (0, 0)),
            pl.BlockSpec((1, hidden), lambda i: (0, 0)),
            pl.BlockSpec((1, hidden), lambda i: (0, 0)),
        ],
        out_specs=pl.BlockSpec((_TB, hidden), lambda i: (blk0 + i, 0)),
        out_shape=jax.ShapeDtypeStruct((n_tok, hidden), jnp.float32),
        input_output_aliases={0: 0},
    )(buf, gathered, tt3, pos_emb, tt_emb, ln_w.reshape(1, hidden), ln_b.reshape(1, hidden))


def kernel(input_ids, token_type_ids, word_emb, token_type_emb, pos_emb, ln_weight, ln_bias):
    batch, seq = input_ids.shape
    hidden = word_emb.shape[1]
    n_tok = batch * seq
    ids = input_ids.reshape(batch, seq).astype(jnp.int32)
    tt_ids = token_type_ids.reshape(batch, seq).astype(jnp.int32)

    gathered = [_sc_gather(word_emb, ids[b]) for b in range(batch)]

    out = jnp.empty((n_tok, hidden), jnp.float32)
    for b in range(batch):
        out = _tc_add_ln_slice(
            out, gathered[b], tt_ids[b], pos_emb, token_type_emb,
            ln_weight, ln_bias, n_tok, b,
        )
    return out.reshape(batch, seq, hidden)


# speedup vs baseline: 1.1354x; 1.1354x over previous
import functools

import jax
import jax.numpy as jnp
from jax import lax
from jax.experimental import pallas as pl
from jax.experimental.pallas import tpu as pltpu
from jax.experimental.pallas import tpu_sc as plsc

EPS = 1e-12

_NC = 2
_NS = 16
_NW = _NC * _NS

_CHUNK = 64

_TB = 256


def _sc_gather(table, ids):
    n_tok = ids.shape[0]
    hidden = table.shape[1]
    per_w = n_tok // _NW
    n_chunks = per_w // _CHUNK

    mesh = plsc.VectorSubcoreMesh(core_axis_name="c", subcore_axis_name="s")

    @functools.partial(
        pl.kernel,
        mesh=mesh,
        out_type=jax.ShapeDtypeStruct((n_tok, hidden), jnp.float32),
        scratch_types=[
            pltpu.VMEM((per_w,), jnp.int32),
            pltpu.VMEM((_CHUNK, hidden), jnp.float32),
            pltpu.VMEM((_CHUNK, hidden), jnp.float32),
            pltpu.SemaphoreType.DMA,
            pltpu.SemaphoreType.DMA,
        ],
    )
    def gather_kernel(table_hbm, ids_hbm, out_hbm, idx_v, buf0, buf1, sem0, sem1):
        wid = lax.axis_index("s") * _NC + lax.axis_index("c")
        base = wid * per_w
        pltpu.sync_copy(ids_hbm.at[pl.ds(base, per_w)], idx_v)
        bufs = (buf0, buf1)
        sems = (sem0, sem1)
        copies = [None] * n_chunks
        copies[0] = pltpu.async_copy(
            table_hbm.at[idx_v.at[pl.ds(0, _CHUNK)]], buf0, sem0
        )
        for k in range(n_chunks):
            nxt = k + 1
            if nxt < n_chunks:
                copies[nxt] = pltpu.async_copy(
                    table_hbm.at[idx_v.at[pl.ds(nxt * _CHUNK, _CHUNK)]],
                    bufs[nxt % 2],
                    sems[nxt % 2],
                )
            copies[k].wait()
            pltpu.sync_copy(bufs[k % 2], out_hbm.at[pl.ds(base + k * _CHUNK, _CHUNK)])

    return gather_kernel(table, ids)


def _ln_body(g_ref, tt_ref, pos_ref, tte_ref, w_ref, b_ref, o_ref):
    x = g_ref[...] + pos_ref[...]
    ttf = tt_ref[0, 0, :].astype(jnp.float32)
    t0 = tte_ref[0, :]
    t1 = tte_ref[1, :]
    x = x + t0[None, :] + ttf[:, None] * (t1 - t0)[None, :]
    u = jnp.mean(x, axis=-1, keepdims=True)
    s = jnp.mean((x - u) ** 2, axis=-1, keepdims=True)
    y = (x - u) * lax.rsqrt(s + EPS)
    o_ref[...] = y * w_ref[0, :][None, :] + b_ref[0, :][None, :]


def _tc_add_ln(gathered, tt_ids, pos_emb, tt_emb, ln_w, ln_b, batch, seq):
    n_tok, hidden = gathered.shape
    sb = seq // _TB

    tt3 = tt_ids.reshape(n_tok // _TB, 1, _TB)

    return pl.pallas_call(
        _ln_body,
        grid=(sb, batch),
        in_specs=[
            pl.BlockSpec((_TB, hidden), lambda i, b: (b * sb + i, 0)),
            pl.BlockSpec((1, 1, _TB), lambda i, b: (b * sb + i, 0, 0)),
            pl.BlockSpec((_TB, hidden), lambda i, b: (i, 0)),
            pl.BlockSpec((2, hidden), lambda i, b: (0, 0)),
            pl.BlockSpec((1, hidden), lambda i, b: (0, 0)),
            pl.BlockSpec((1, hidden), lambda i, b: (0, 0)),
        ],
        out_specs=pl.BlockSpec((_TB, hidden), lambda i, b: (b * sb + i, 0)),
        out_shape=jax.ShapeDtypeStruct((n_tok, hidden), jnp.float32),
    )(gathered, tt3, pos_emb, tt_emb, ln_w.reshape(1, hidden), ln_b.reshape(1, hidden))


def kernel(input_ids, token_type_ids, word_emb, token_type_emb, pos_emb, ln_weight, ln_bias):
    batch, seq = input_ids.shape
    hidden = word_emb.shape[1]
    ids = input_ids.reshape(-1).astype(jnp.int32)
    tt_ids = token_type_ids.reshape(-1).astype(jnp.int32)
    gathered = _sc_gather(word_emb, ids)
    out = _tc_add_ln(gathered, tt_ids, pos_emb, token_type_emb, ln_weight, ln_bias, batch, seq)
    return out.reshape(batch, seq, hidden)


# speedup vs baseline: 1.3127x; 1.1562x over previous
import functools

import jax
import jax.numpy as jnp
from jax import lax
from jax.experimental import pallas as pl
from jax.experimental.pallas import tpu as pltpu
from jax.experimental.pallas import tpu_sc as plsc

EPS = 1e-12

_NC = 2
_NS = 16
_NW = _NC * _NS

_CHUNK = 64

_TB = 512


def _sc_gather(table, ids):
    n_tok = ids.shape[0]
    hidden = table.shape[1]
    per_w = n_tok // _NW
    n_chunks = per_w // _CHUNK

    mesh = plsc.VectorSubcoreMesh(core_axis_name="c", subcore_axis_name="s")

    @functools.partial(
        pl.kernel,
        mesh=mesh,
        out_type=jax.ShapeDtypeStruct((n_tok, hidden), jnp.float32),
        scratch_types=[
            pltpu.VMEM((per_w,), jnp.int32),
            pltpu.VMEM((_CHUNK, hidden), jnp.float32),
            pltpu.VMEM((_CHUNK, hidden), jnp.float32),
            pltpu.SemaphoreType.DMA,
            pltpu.SemaphoreType.DMA,
        ],
    )
    def gather_kernel(table_hbm, ids_hbm, out_hbm, idx_v, buf0, buf1, sem0, sem1):
        wid = lax.axis_index("s") * _NC + lax.axis_index("c")
        base = wid * per_w
        pltpu.sync_copy(ids_hbm.at[pl.ds(base, per_w)], idx_v)
        bufs = (buf0, buf1)
        sems = (sem0, sem1)
        copies = [None] * n_chunks
        copies[0] = pltpu.async_copy(
            table_hbm.at[idx_v.at[pl.ds(0, _CHUNK)]], buf0, sem0
        )
        for k in range(n_chunks):
            nxt = k + 1
            if nxt < n_chunks:
                copies[nxt] = pltpu.async_copy(
                    table_hbm.at[idx_v.at[pl.ds(nxt * _CHUNK, _CHUNK)]],
                    bufs[nxt % 2],
                    sems[nxt % 2],
                )
            copies[k].wait()
            pltpu.sync_copy(bufs[k % 2], out_hbm.at[pl.ds(base + k * _CHUNK, _CHUNK)])

    return gather_kernel(table, ids)


def _ln_body(g_ref, tt_ref, pos_ref, tte_ref, w_ref, b_ref, o_ref):
    x = g_ref[...] + pos_ref[...]
    ttf = tt_ref[0, 0, :].astype(jnp.float32)
    t0 = tte_ref[0, :]
    t1 = tte_ref[1, :]
    x = x + t0[None, :] + ttf[:, None] * (t1 - t0)[None, :]
    u = jnp.mean(x, axis=-1, keepdims=True)
    s = jnp.mean((x - u) ** 2, axis=-1, keepdims=True)
    y = (x - u) * lax.rsqrt(s + EPS)
    o_ref[...] = y * w_ref[0, :][None, :] + b_ref[0, :][None, :]


def _tc_add_ln(gathered, tt_ids, pos_emb, tt_emb, ln_w, ln_b, batch, seq):
    n_tok, hidden = gathered.shape
    sb = seq // _TB

    tt3 = tt_ids.reshape(n_tok // _TB, 1, _TB)

    return pl.pallas_call(
        _ln_body,
        grid=(sb, batch),
        in_specs=[
            pl.BlockSpec((_TB, hidden), lambda i, b: (b * sb + i, 0)),
            pl.BlockSpec((1, 1, _TB), lambda i, b: (b * sb + i, 0, 0)),
            pl.BlockSpec((_TB, hidden), lambda i, b: (i, 0)),
            pl.BlockSpec((2, hidden), lambda i, b: (0, 0)),
            pl.BlockSpec((1, hidden), lambda i, b: (0, 0)),
            pl.BlockSpec((1, hidden), lambda i, b: (0, 0)),
        ],
        out_specs=pl.BlockSpec((_TB, hidden), lambda i, b: (b * sb + i, 0)),
        out_shape=jax.ShapeDtypeStruct((n_tok, hidden), jnp.float32),
    )(gathered, tt3, pos_emb, tt_emb, ln_w.reshape(1, hidden), ln_b.reshape(1, hidden))


def kernel(input_ids, token_type_ids, word_emb, token_type_emb, pos_emb, ln_weight, ln_bias):
    batch, seq = input_ids.shape
    hidden = word_emb.shape[1]
    ids = input_ids.reshape(-1).astype(jnp.int32)
    tt_ids = token_type_ids.reshape(-1).astype(jnp.int32)
    gathered = _sc_gather(word_emb, ids)
    out = _tc_add_ln(gathered, tt_ids, pos_emb, token_type_emb, ln_weight, ln_bias, batch, seq)
    return out.reshape(batch, seq, hidden)


# speedup vs baseline: 1.3846x; 1.0547x over previous
import functools

import jax
import jax.numpy as jnp
from jax import lax
from jax.experimental import pallas as pl
from jax.experimental.pallas import tpu as pltpu
from jax.experimental.pallas import tpu_sc as plsc

EPS = 1e-12

_NC = 2
_NS = 16
_NW = _NC * _NS

_CHUNK = 64

_TB = 1024


def _sc_gather(table, ids):
    n_tok = ids.shape[0]
    hidden = table.shape[1]
    per_w = n_tok // _NW
    n_chunks = per_w // _CHUNK

    mesh = plsc.VectorSubcoreMesh(core_axis_name="c", subcore_axis_name="s")

    @functools.partial(
        pl.kernel,
        mesh=mesh,
        out_type=jax.ShapeDtypeStruct((n_tok, hidden), jnp.float32),
        scratch_types=[
            pltpu.VMEM((per_w,), jnp.int32),
            pltpu.VMEM((_CHUNK, hidden), jnp.float32),
            pltpu.VMEM((_CHUNK, hidden), jnp.float32),
            pltpu.SemaphoreType.DMA,
            pltpu.SemaphoreType.DMA,
        ],
    )
    def gather_kernel(table_hbm, ids_hbm, out_hbm, idx_v, buf0, buf1, sem0, sem1):
        wid = lax.axis_index("s") * _NC + lax.axis_index("c")
        base = wid * per_w
        pltpu.sync_copy(ids_hbm.at[pl.ds(base, per_w)], idx_v)
        bufs = (buf0, buf1)
        sems = (sem0, sem1)
        copies = [None] * n_chunks
        copies[0] = pltpu.async_copy(
            table_hbm.at[idx_v.at[pl.ds(0, _CHUNK)]], buf0, sem0
        )
        for k in range(n_chunks):
            nxt = k + 1
            if nxt < n_chunks:
                copies[nxt] = pltpu.async_copy(
                    table_hbm.at[idx_v.at[pl.ds(nxt * _CHUNK, _CHUNK)]],
                    bufs[nxt % 2],
                    sems[nxt % 2],
                )
            copies[k].wait()
            pltpu.sync_copy(bufs[k % 2], out_hbm.at[pl.ds(base + k * _CHUNK, _CHUNK)])

    return gather_kernel(table, ids)


def _ln_body(g_ref, tt_ref, pos_ref, tte_ref, w_ref, b_ref, o_ref):
    x = g_ref[...] + pos_ref[...]
    ttf = tt_ref[0, 0, :].astype(jnp.float32)
    t0 = tte_ref[0, :]
    t1 = tte_ref[1, :]
    x = x + t0[None, :] + ttf[:, None] * (t1 - t0)[None, :]
    u = jnp.mean(x, axis=-1, keepdims=True)
    s = jnp.mean((x - u) ** 2, axis=-1, keepdims=True)
    y = (x - u) * lax.rsqrt(s + EPS)
    o_ref[...] = y * w_ref[0, :][None, :] + b_ref[0, :][None, :]


def _tc_add_ln(gathered, tt_ids, pos_emb, tt_emb, ln_w, ln_b, batch, seq):
    n_tok, hidden = gathered.shape
    sb = seq // _TB

    tt3 = tt_ids.reshape(n_tok // _TB, 1, _TB)

    return pl.pallas_call(
        _ln_body,
        grid=(sb, batch),
        in_specs=[
            pl.BlockSpec((_TB, hidden), lambda i, b: (b * sb + i, 0)),
            pl.BlockSpec((1, 1, _TB), lambda i, b: (b * sb + i, 0, 0)),
            pl.BlockSpec((_TB, hidden), lambda i, b: (i, 0)),
            pl.BlockSpec((2, hidden), lambda i, b: (0, 0)),
            pl.BlockSpec((1, hidden), lambda i, b: (0, 0)),
            pl.BlockSpec((1, hidden), lambda i, b: (0, 0)),
        ],
        out_specs=pl.BlockSpec((_TB, hidden), lambda i, b: (b * sb + i, 0)),
        out_shape=jax.ShapeDtypeStruct((n_tok, hidden), jnp.float32),
    )(gathered, tt3, pos_emb, tt_emb, ln_w.reshape(1, hidden), ln_b.reshape(1, hidden))


def kernel(input_ids, token_type_ids, word_emb, token_type_emb, pos_emb, ln_weight, ln_bias):
    batch, seq = input_ids.shape
    hidden = word_emb.shape[1]
    ids = input_ids.reshape(-1).astype(jnp.int32)
    tt_ids = token_type_ids.reshape(-1).astype(jnp.int32)
    gathered = _sc_gather(word_emb, ids)
    out = _tc_add_ln(gathered, tt_ids, pos_emb, token_type_emb, ln_weight, ln_bias, batch, seq)
    return out.reshape(batch, seq, hidden)


# speedup vs baseline: 1.4249x; 1.0291x over previous
import functools

import jax
import jax.numpy as jnp
from jax import lax
from jax.experimental import pallas as pl
from jax.experimental.pallas import tpu as pltpu
from jax.experimental.pallas import tpu_sc as plsc

EPS = 1e-12

_NC = 2
_NS = 16
_NW = _NC * _NS

_CHUNK = 64

_TB = 2048


def _sc_gather(table, ids):
    n_tok = ids.shape[0]
    hidden = table.shape[1]
    per_w = n_tok // _NW
    n_chunks = per_w // _CHUNK

    mesh = plsc.VectorSubcoreMesh(core_axis_name="c", subcore_axis_name="s")

    @functools.partial(
        pl.kernel,
        mesh=mesh,
        out_type=jax.ShapeDtypeStruct((n_tok, hidden), jnp.float32),
        scratch_types=[
            pltpu.VMEM((per_w,), jnp.int32),
            pltpu.VMEM((_CHUNK, hidden), jnp.float32),
            pltpu.VMEM((_CHUNK, hidden), jnp.float32),
            pltpu.SemaphoreType.DMA,
            pltpu.SemaphoreType.DMA,
        ],
    )
    def gather_kernel(table_hbm, ids_hbm, out_hbm, idx_v, buf0, buf1, sem0, sem1):
        wid = lax.axis_index("s") * _NC + lax.axis_index("c")
        base = wid * per_w
        pltpu.sync_copy(ids_hbm.at[pl.ds(base, per_w)], idx_v)
        bufs = (buf0, buf1)
        sems = (sem0, sem1)
        copies = [None] * n_chunks
        copies[0] = pltpu.async_copy(
            table_hbm.at[idx_v.at[pl.ds(0, _CHUNK)]], buf0, sem0
        )
        for k in range(n_chunks):
            nxt = k + 1
            if nxt < n_chunks:
                copies[nxt] = pltpu.async_copy(
                    table_hbm.at[idx_v.at[pl.ds(nxt * _CHUNK, _CHUNK)]],
                    bufs[nxt % 2],
                    sems[nxt % 2],
                )
            copies[k].wait()
            pltpu.sync_copy(bufs[k % 2], out_hbm.at[pl.ds(base + k * _CHUNK, _CHUNK)])

    return gather_kernel(table, ids)


def _ln_body(g_ref, tt_ref, pos_ref, tte_ref, w_ref, b_ref, o_ref):
    x = g_ref[...] + pos_ref[...]
    ttf = tt_ref[0, 0, :].astype(jnp.float32)
    t0 = tte_ref[0, :]
    t1 = tte_ref[1, :]
    x = x + t0[None, :] + ttf[:, None] * (t1 - t0)[None, :]
    u = jnp.mean(x, axis=-1, keepdims=True)
    s = jnp.mean((x - u) ** 2, axis=-1, keepdims=True)
    y = (x - u) * lax.rsqrt(s + EPS)
    o_ref[...] = y * w_ref[0, :][None, :] + b_ref[0, :][None, :]


def _tc_add_ln(gathered, tt_ids, pos_emb, tt_emb, ln_w, ln_b, batch, seq):
    n_tok, hidden = gathered.shape
    sb = seq // _TB

    tt3 = tt_ids.reshape(n_tok // _TB, 1, _TB)

    return pl.pallas_call(
        _ln_body,
        grid=(sb, batch),
        in_specs=[
            pl.BlockSpec((_TB, hidden), lambda i, b: (b * sb + i, 0)),
            pl.BlockSpec((1, 1, _TB), lambda i, b: (b * sb + i, 0, 0)),
            pl.BlockSpec((_TB, hidden), lambda i, b: (i, 0)),
            pl.BlockSpec((2, hidden), lambda i, b: (0, 0)),
            pl.BlockSpec((1, hidden), lambda i, b: (0, 0)),
            pl.BlockSpec((1, hidden), lambda i, b: (0, 0)),
        ],
        out_specs=pl.BlockSpec((_TB, hidden), lambda i, b: (b * sb + i, 0)),
        out_shape=jax.ShapeDtypeStruct((n_tok, hidden), jnp.float32),
    )(gathered, tt3, pos_emb, tt_emb, ln_w.reshape(1, hidden), ln_b.reshape(1, hidden))


def kernel(input_ids, token_type_ids, word_emb, token_type_emb, pos_emb, ln_weight, ln_bias):
    batch, seq = input_ids.shape
    hidden = word_emb.shape[1]
    ids = input_ids.reshape(-1).astype(jnp.int32)
    tt_ids = token_type_ids.reshape(-1).astype(jnp.int32)
    gathered = _sc_gather(word_emb, ids)
    out = _tc_add_ln(gathered, tt_ids, pos_emb, token_type_emb, ln_weight, ln_bias, batch, seq)
    return out.reshape(batch, seq, hidden)
